# Initial kernel scaffold; baseline (speedup 1.0000x reference)
import jax
import jax.numpy as jnp
from jax.experimental import pallas as pl


def kernel(x, edge_index, edge_type, basis0, comp0, root0, bias0, basis1, comp1, root1, bias1, basis2, comp2, root2, bias2):
    raise NotImplementedError("write your pallas kernel here")


# speedup vs baseline: 4.5867x; 4.5867x over previous
import functools

import jax
import jax.numpy as jnp
from jax import lax
from jax.experimental import pallas as pl
from jax.experimental.pallas import tpu as pltpu
from jax.experimental.pallas import tpu_sc as plsc

N = 10000
E = 320000
D = 128
R = 6
NB = 30

CH = 4
CW = 32
RN = R * N
RNP = 60032
STRIPE = RNP // 16
NSUB = 16
NCORE = 2
ES = E // NSUB
K = 80
NBATCH = ES // K

BN = 1000
NBLK = N // BN


@functools.lru_cache(maxsize=None)
def _make_sc_agg(nc):
    mesh = plsc.VectorSubcoreMesh(core_axis_name="c", subcore_axis_name="s")

    @functools.partial(
        pl.kernel,
        out_type=jax.ShapeDtypeStruct((nc * RNP, CW), jnp.float32),
        mesh=mesh,
        compiler_params=pltpu.CompilerParams(use_tc_tiling_on_sc=False),
        scratch_types=[
            pltpu.VMEM((K,), jnp.int32),
            pltpu.VMEM((K,), jnp.int32),
            pltpu.VMEM((K, CW), jnp.float32),
            pltpu.VMEM_SHARED((RNP, CW), jnp.float32),
            pltpu.SemaphoreType.DMA,
        ],
    )
    def sc_agg(xflat, src4, eidx, zstripe, onesr, out,
               srcbuf, eidxbuf, rowbuf, acc, sem):
        core = lax.axis_index("c")
        sub = lax.axis_index("s")
        row0 = sub * STRIPE
        for c in range(nc):
            @pl.when(core == (c % NCORE))
            def _(c=c):
                pltpu.sync_copy(zstripe, acc.at[pl.ds(row0, STRIPE)])
                if c == CH:
                    pltpu.sync_copy(onesr, rowbuf)
                plsc.subcore_barrier()

                def step(b, carry):
                    base = sub * ES + b * K
                    pltpu.sync_copy(eidx.at[pl.ds(base, K)], eidxbuf)
                    if c < CH:
                        pltpu.sync_copy(src4.at[pl.ds(c * E + base, K)],
                                        srcbuf)
                        pltpu.async_copy(xflat.at[srcbuf], rowbuf, sem).wait()
                    pltpu.sync_copy(rowbuf, acc.at[eidxbuf], add=True)
                    return carry

                lax.fori_loop(0, NBATCH, step, 0)
                plsc.subcore_barrier()
                pltpu.sync_copy(acc.at[pl.ds(row0, STRIPE)],
                                out.at[pl.ds(c * RNP + row0, STRIPE)])
                plsc.subcore_barrier()

    return sc_agg


def _sc_call(nc, *args):
    return _make_sc_agg(nc)(*args)


def _wprep(comp, basis, dout):
    din = basis.shape[1]

    def body(c_ref, b_ref, o_ref):
        o_ref[...] = jnp.dot(c_ref[...], b_ref[...],
                             preferred_element_type=jnp.float32)

    flat = pl.pallas_call(
        body,
        out_shape=jax.ShapeDtypeStruct((R, din * dout), jnp.float32),
    )(comp, basis.reshape(NB, din * dout))
    return flat.reshape(R, din, dout)


def _make_layer_tc(din, dout, act):
    def body(x_ref, acc_ref, cnt_ref, w_ref, root_ref, bias_ref, o_ref):
        r = pl.program_id(1)
        contrib = jnp.zeros((BN, dout), jnp.float32)
        for c in range(CH):
            contrib += jnp.dot(acc_ref[c], w_ref[0, c * CW:(c + 1) * CW, :],
                               preferred_element_type=jnp.float32)
        cnt = jnp.maximum(cnt_ref[0, 0, 0, :], 1.0)
        contrib = contrib / cnt[:, None]

        @pl.when(r == 0)
        def _():
            o_ref[...] = (jnp.dot(x_ref[...], root_ref[...],
                                  preferred_element_type=jnp.float32)
                          + bias_ref[...] + contrib)

        @pl.when(r > 0)
        def _():
            o_ref[...] = o_ref[...] + contrib

        @pl.when(r == R - 1)
        def _():
            h = o_ref[...]
            if act == "relu":
                o_ref[...] = jnp.maximum(h, 0.0)
            else:
                col = lax.broadcasted_iota(jnp.int32, (BN, dout), 1)
                hm = jnp.where(col < 2, h, -jnp.inf)
                m = jnp.max(hm, axis=1, keepdims=True)
                lse = m + jnp.log(jnp.sum(jnp.exp(hm - m), axis=1,
                                          keepdims=True))
                o_ref[...] = h - lse

    return pl.pallas_call(
        body,
        grid=(NBLK, R),
        in_specs=[
            pl.BlockSpec((BN, din), lambda i, r: (i, 0)),
            pl.BlockSpec((CH, BN, CW), lambda i, r: (0, r * NBLK + i, 0)),
            pl.BlockSpec((1, 1, 1, BN), lambda i, r: (r, i, 0, 0)),
            pl.BlockSpec((1, din, dout), lambda i, r: (r, 0, 0)),
            pl.BlockSpec((din, dout), lambda i, r: (0, 0)),
            pl.BlockSpec((1, dout), lambda i, r: (0, 0)),
        ],
        out_specs=pl.BlockSpec((BN, dout), lambda i, r: (i, 0)),
        out_shape=jax.ShapeDtypeStruct((N, dout), jnp.float32),
    )


_layer_relu = _make_layer_tc(D, D, "relu")
_layer_lsm = _make_layer_tc(D, D, "logsoftmax")


def _xflat(x):
    return x.reshape(N, CH, CW).transpose(1, 0, 2).reshape(CH * N, CW)


def kernel(x, edge_index, edge_type, basis0, comp0, root0, bias0,
           basis1, comp1, root1, bias1, basis2, comp2, root2, bias2):
    src = edge_index[0]
    dst = edge_index[1]
    eidx = edge_type * N + dst
    offs = jnp.arange(CH, dtype=jnp.int32) * N
    src4 = (src[None, :] + offs[:, None]).reshape(CH * E)
    zstripe = jnp.zeros((STRIPE, CW), jnp.float32)
    onesr = jnp.ones((K, CW), jnp.float32)

    a = _sc_call(5, _xflat(x), src4, eidx, zstripe, onesr).reshape(5, RNP, CW)
    cnt4 = a[CH, :RN, 0].reshape(R, NBLK, 1, BN)
    w = _wprep(comp0, basis0, D)
    h = _layer_relu(x, a[:CH], cnt4, w, root0, bias0.reshape(1, D))

    a = _sc_call(4, _xflat(h), src4, eidx, zstripe, onesr).reshape(CH, RNP, CW)
    w = _wprep(comp1, basis1, D)
    h = _layer_relu(h, a, cnt4, w, root1, bias1.reshape(1, D))

    a = _sc_call(4, _xflat(h), src4, eidx, zstripe, onesr).reshape(CH, RNP, CW)
    w = _wprep(comp2, basis2, 2)
    wp = jnp.pad(w, ((0, 0), (0, 0), (0, D - 2)))
    rootp = jnp.pad(root2, ((0, 0), (0, D - 2)))
    biasp = jnp.pad(bias2, (0, D - 2)).reshape(1, D)
    out = _layer_lsm(h, a, cnt4, wp, rootp, biasp)
    return out[:, :2]


# speedup vs baseline: 7.8143x; 1.7037x over previous
import functools

import jax
import jax.numpy as jnp
from jax import lax
from jax.experimental import pallas as pl
from jax.experimental.pallas import tpu as pltpu
from jax.experimental.pallas import tpu_sc as plsc

N = 10000
E = 320000
D = 128
R = 6
NB = 30

CH = 8
CW = 16
RN = R * N
RNP = 60032
STRIPE = RNP // 16
NSUB = 16
NCORE = 2
ES = E // NSUB
K = 100
NBATCH = ES // K

BN = 1000
NBLK = N // BN


NBUF = 4
GOUT = NBATCH // NBUF


@functools.lru_cache(maxsize=None)
def _make_sc_agg(with_counts):
    nout = CH + 2 if with_counts else CH
    mesh = plsc.VectorSubcoreMesh(core_axis_name="c", subcore_axis_name="s")

    @functools.partial(
        pl.kernel,
        out_type=jax.ShapeDtypeStruct((nout * RNP, CW), jnp.float32),
        mesh=mesh,
        compiler_params=pltpu.CompilerParams(use_tc_tiling_on_sc=False),
        scratch_types=[
            pltpu.VMEM((NBATCH, K), jnp.int32),
            pltpu.VMEM((NBATCH, K), jnp.int32),
            pltpu.VMEM((NBUF, K, CW), jnp.float32),
            pltpu.VMEM_SHARED((RNP, CW), jnp.float32),
            pltpu.SemaphoreType.DMA((NBUF,)),
        ],
    )
    def sc_agg(xflat, src4, eidx, zstripe, onesr, out,
               eidx_vm, src_vm, rowbuf, acc, sems):
        core = lax.axis_index("c")
        sub = lax.axis_index("s")
        row0 = sub * STRIPE
        pltpu.sync_copy(eidx.at[sub], eidx_vm)
        for c in range(CH):
            @pl.when(core == (c % NCORE))
            def _(c=c):
                pltpu.sync_copy(zstripe, acc.at[pl.ds(row0, STRIPE)])
                pltpu.sync_copy(src4.at[c].at[sub], src_vm)
                plsc.subcore_barrier()

                def fire(b, j):
                    pltpu.async_copy(xflat.at[src_vm.at[b]],
                                     rowbuf.at[j], sems.at[j])

                for j in range(NBUF):
                    fire(j, j)

                def step(g, carry):
                    for j in range(NBUF):
                        b = g * NBUF + j
                        pltpu.make_async_copy(
                            xflat.at[src_vm.at[b]], rowbuf.at[j],
                            sems.at[j]).wait()
                        pltpu.sync_copy(rowbuf.at[j],
                                        acc.at[eidx_vm.at[b]], add=True)

                        @pl.when(g < GOUT - 1)
                        def _(b=b, j=j):
                            fire(b + NBUF, j)
                    return carry

                lax.fori_loop(0, GOUT, step, 0)
                plsc.subcore_barrier()
                pltpu.sync_copy(acc.at[pl.ds(row0, STRIPE)],
                                out.at[pl.ds(c * RNP + row0, STRIPE)])
                plsc.subcore_barrier()
        if with_counts:
            pltpu.sync_copy(zstripe, acc.at[pl.ds(row0, STRIPE)])
            pltpu.sync_copy(onesr, rowbuf.at[0])
            plsc.subcore_barrier()
            half = NBATCH // NCORE
            b0 = core * half

            def cstep(i, carry):
                pltpu.sync_copy(rowbuf.at[0],
                                acc.at[eidx_vm.at[b0 + i]], add=True)
                return carry

            lax.fori_loop(0, half, cstep, 0)
            plsc.subcore_barrier()
            cblk = (CH + core) * RNP
            pltpu.sync_copy(acc.at[pl.ds(row0, STRIPE)],
                            out.at[pl.ds(cblk + row0, STRIPE)])

    return sc_agg


def _sc_call(nc, *args):
    return _make_sc_agg(nc)(*args)


def _wprep(comp, basis, dout):
    din = basis.shape[1]

    def body(c_ref, b_ref, o_ref):
        o_ref[...] = jnp.dot(c_ref[...], b_ref[...],
                             preferred_element_type=jnp.float32)

    flat = pl.pallas_call(
        body,
        out_shape=jax.ShapeDtypeStruct((R, din * dout), jnp.float32),
    )(comp, basis.reshape(NB, din * dout))
    return flat.reshape(R, din, dout)


def _make_layer_tc(din, dout, act):
    def body(x_ref, acc_ref, cnta_ref, cntb_ref, w_ref, root_ref, bias_ref,
             o_ref):
        r = pl.program_id(1)
        contrib = jnp.zeros((BN, dout), jnp.float32)
        for c in range(CH):
            contrib += jnp.dot(acc_ref[c], w_ref[0, c * CW:(c + 1) * CW, :],
                               preferred_element_type=jnp.float32)
        cnt = jnp.maximum(cnta_ref[0, 0, 0, :] + cntb_ref[0, 0, 0, :], 1.0)
        contrib = contrib / cnt[:, None]

        @pl.when(r == 0)
        def _():
            o_ref[...] = (jnp.dot(x_ref[...], root_ref[...],
                                  preferred_element_type=jnp.float32)
                          + bias_ref[...] + contrib)

        @pl.when(r > 0)
        def _():
            o_ref[...] = o_ref[...] + contrib

        @pl.when(r == R - 1)
        def _():
            h = o_ref[...]
            if act == "relu":
                o_ref[...] = jnp.maximum(h, 0.0)
            else:
                col = lax.broadcasted_iota(jnp.int32, (BN, dout), 1)
                hm = jnp.where(col < 2, h, -jnp.inf)
                m = jnp.max(hm, axis=1, keepdims=True)
                lse = m + jnp.log(jnp.sum(jnp.exp(hm - m), axis=1,
                                          keepdims=True))
                o_ref[...] = h - lse

    return pl.pallas_call(
        body,
        grid=(NBLK, R),
        in_specs=[
            pl.BlockSpec((BN, din), lambda i, r: (i, 0)),
            pl.BlockSpec((CH, BN, CW), lambda i, r: (0, r * NBLK + i, 0)),
            pl.BlockSpec((1, 1, 1, BN), lambda i, r: (r, i, 0, 0)),
            pl.BlockSpec((1, 1, 1, BN), lambda i, r: (r, i, 0, 0)),
            pl.BlockSpec((1, din, dout), lambda i, r: (r, 0, 0)),
            pl.BlockSpec((din, dout), lambda i, r: (0, 0)),
            pl.BlockSpec((1, dout), lambda i, r: (0, 0)),
        ],
        out_specs=pl.BlockSpec((BN, dout), lambda i, r: (i, 0)),
        out_shape=jax.ShapeDtypeStruct((N, dout), jnp.float32),
    )


_layer_relu = _make_layer_tc(D, D, "relu")
_layer_lsm = _make_layer_tc(D, D, "logsoftmax")


def _xflat(x):
    return x.reshape(N, CH, CW).transpose(1, 0, 2).reshape(CH * N, CW)


def kernel(x, edge_index, edge_type, basis0, comp0, root0, bias0,
           basis1, comp1, root1, bias1, basis2, comp2, root2, bias2):
    src = edge_index[0]
    dst = edge_index[1]
    eidx = (edge_type * N + dst).reshape(NSUB, NBATCH, K)
    offs = jnp.arange(CH, dtype=jnp.int32) * N
    src4 = (src[None, :] + offs[:, None]).reshape(CH, NSUB, NBATCH, K)
    zstripe = jnp.zeros((STRIPE, CW), jnp.float32)
    onesr = jnp.ones((K, CW), jnp.float32)

    a = _sc_call(True, _xflat(x), src4, eidx, zstripe,
                 onesr).reshape(CH + 2, RNP, CW)
    cnt4a = a[CH, :RN, 0].reshape(R, NBLK, 1, BN)
    cnt4b = a[CH + 1, :RN, 0].reshape(R, NBLK, 1, BN)
    w = _wprep(comp0, basis0, D)
    h = _layer_relu(x, a[:CH], cnt4a, cnt4b, w, root0, bias0.reshape(1, D))

    a = _sc_call(False, _xflat(h), src4, eidx, zstripe,
                 onesr).reshape(CH, RNP, CW)
    w = _wprep(comp1, basis1, D)
    h = _layer_relu(h, a, cnt4a, cnt4b, w, root1, bias1.reshape(1, D))

    a = _sc_call(False, _xflat(h), src4, eidx, zstripe,
                 onesr).reshape(CH, RNP, CW)
    w = _wprep(comp2, basis2, 2)
    wp = jnp.pad(w, ((0, 0), (0, 0), (0, D - 2)))
    rootp = jnp.pad(root2, ((0, 0), (0, D - 2)))
    biasp = jnp.pad(bias2, (0, D - 2)).reshape(1, D)
    out = _layer_lsm(h, a, cnt4a, cnt4b, wp, rootp, biasp)
    return out[:, :2]


# speedup vs baseline: 12.2106x; 1.5626x over previous
import functools

import jax
import jax.numpy as jnp
from jax import lax
from jax.experimental import pallas as pl
from jax.experimental.pallas import tpu as pltpu
from jax.experimental.pallas import tpu_sc as plsc

N = 10000
E = 320000
D = 128
R = 6
NB = 30

CH = 8
CW = 16
RN = R * N
RNP = 60032
STRIPE = RNP // 16
NSUB = 16
NCORE = 2
ES = E // NSUB
K = 100
NBATCH = ES // K

BN = 1000
NBLK = N // BN


NBUF = 4
GOUT = NBATCH // NBUF


@functools.lru_cache(maxsize=None)
def _make_sc_agg(with_counts):
    out_type = [jax.ShapeDtypeStruct((RNP, D), jnp.float32)]
    if with_counts:
        out_type.append(
            jax.ShapeDtypeStruct((NCORE * RNP, CW), jnp.float32))
    mesh = plsc.VectorSubcoreMesh(core_axis_name="c", subcore_axis_name="s")

    @functools.partial(
        pl.kernel,
        out_type=out_type,
        mesh=mesh,
        compiler_params=pltpu.CompilerParams(use_tc_tiling_on_sc=False),
        scratch_types=[
            pltpu.VMEM((NBATCH, K), jnp.int32),
            pltpu.VMEM((NBATCH, K), jnp.int32),
            pltpu.VMEM((NBUF, K, CW), jnp.float32),
            pltpu.VMEM_SHARED((RNP, CW), jnp.float32),
            pltpu.SemaphoreType.DMA((NBUF,)),
        ],
    )
    def sc_agg(xflat, src4, eidx, zstripe, onesr, *out_and_scratch):
        if with_counts:
            out, cout = out_and_scratch[:2]
            eidx_vm, src_vm, rowbuf, acc, sems = out_and_scratch[2:]
        else:
            out = out_and_scratch[0]
            eidx_vm, src_vm, rowbuf, acc, sems = out_and_scratch[1:]
        core = lax.axis_index("c")
        sub = lax.axis_index("s")
        row0 = sub * STRIPE
        pltpu.sync_copy(eidx.at[sub], eidx_vm)
        for c in range(CH):
            @pl.when(core == (c % NCORE))
            def _(c=c):
                pltpu.sync_copy(zstripe, acc.at[pl.ds(row0, STRIPE)])
                pltpu.sync_copy(src4.at[c].at[sub], src_vm)
                plsc.subcore_barrier()

                def fire(b, j):
                    pltpu.async_copy(xflat.at[src_vm.at[b]],
                                     rowbuf.at[j], sems.at[j])

                for j in range(NBUF):
                    fire(j, j)

                def step(g, carry):
                    for j in range(NBUF):
                        b = g * NBUF + j
                        pltpu.make_async_copy(
                            xflat.at[src_vm.at[b]], rowbuf.at[j],
                            sems.at[j]).wait()
                        pltpu.sync_copy(rowbuf.at[j],
                                        acc.at[eidx_vm.at[b]], add=True)

                        @pl.when(g < GOUT - 1)
                        def _(b=b, j=j):
                            fire(b + NBUF, j)
                    return carry

                lax.fori_loop(0, GOUT, step, 0)
                plsc.subcore_barrier()
                pltpu.sync_copy(
                    acc.at[pl.ds(row0, STRIPE)],
                    out.at[pl.ds(row0, STRIPE), pl.ds(c * CW, CW)])
                plsc.subcore_barrier()
        if with_counts:
            pltpu.sync_copy(zstripe, acc.at[pl.ds(row0, STRIPE)])
            pltpu.sync_copy(onesr, rowbuf.at[0])
            plsc.subcore_barrier()
            half = NBATCH // NCORE
            b0 = core * half

            def cstep(i, carry):
                pltpu.sync_copy(rowbuf.at[0],
                                acc.at[eidx_vm.at[b0 + i]], add=True)
                return carry

            lax.fori_loop(0, half, cstep, 0)
            plsc.subcore_barrier()
            pltpu.sync_copy(acc.at[pl.ds(row0, STRIPE)],
                            cout.at[pl.ds(core * RNP + row0, STRIPE)])

    return sc_agg


def _sc_call(with_counts, *args):
    return _make_sc_agg(with_counts)(*args)


def _wprep(comp, basis, dout):
    din = basis.shape[1]

    def body(c_ref, b_ref, o_ref):
        o_ref[...] = jnp.dot(c_ref[...], b_ref[...],
                             preferred_element_type=jnp.float32)

    flat = pl.pallas_call(
        body,
        out_shape=jax.ShapeDtypeStruct((R, din * dout), jnp.float32),
    )(comp, basis.reshape(NB, din * dout))
    return flat.reshape(R, din, dout)


def _make_layer_tc(din, dout, act, want_flat):
    def body(x_ref, acc_ref, cnta_ref, cntb_ref, w_ref, root_ref, bias_ref,
             *o_refs):
        o_ref = o_refs[0]
        r = pl.program_id(1)
        contrib = jnp.dot(acc_ref[...], w_ref[0],
                          preferred_element_type=jnp.float32)
        cnt = jnp.maximum(cnta_ref[0, 0, 0, :] + cntb_ref[0, 0, 0, :], 1.0)
        contrib = contrib / cnt[:, None]

        @pl.when(r == 0)
        def _():
            o_ref[...] = (jnp.dot(x_ref[...], root_ref[...],
                                  preferred_element_type=jnp.float32)
                          + bias_ref[...] + contrib)

        @pl.when(r > 0)
        def _():
            o_ref[...] = o_ref[...] + contrib

        @pl.when(r == R - 1)
        def _():
            h = o_ref[...]
            if act == "relu":
                h = jnp.maximum(h, 0.0)
                o_ref[...] = h
            else:
                col = lax.broadcasted_iota(jnp.int32, (BN, dout), 1)
                hm = jnp.where(col < 2, h, -jnp.inf)
                m = jnp.max(hm, axis=1, keepdims=True)
                lse = m + jnp.log(jnp.sum(jnp.exp(hm - m), axis=1,
                                          keepdims=True))
                o_ref[...] = h - lse
            if want_flat:
                for c in range(CH):
                    o_refs[1][c] = h[:, c * CW:(c + 1) * CW]

    out_specs = [pl.BlockSpec((BN, dout), lambda i, r: (i, 0))]
    out_shape = [jax.ShapeDtypeStruct((N, dout), jnp.float32)]
    if want_flat:
        out_specs.append(pl.BlockSpec((CH, BN, CW), lambda i, r: (0, i, 0)))
        out_shape.append(jax.ShapeDtypeStruct((CH, N, CW), jnp.float32))
    return pl.pallas_call(
        body,
        grid=(NBLK, R),
        in_specs=[
            pl.BlockSpec((BN, din), lambda i, r: (i, 0)),
            pl.BlockSpec((BN, din), lambda i, r: (r * NBLK + i, 0)),
            pl.BlockSpec((1, 1, 1, BN), lambda i, r: (r, i, 0, 0)),
            pl.BlockSpec((1, 1, 1, BN), lambda i, r: (r, i, 0, 0)),
            pl.BlockSpec((1, din, dout), lambda i, r: (r, 0, 0)),
            pl.BlockSpec((din, dout), lambda i, r: (0, 0)),
            pl.BlockSpec((1, dout), lambda i, r: (0, 0)),
        ],
        out_specs=out_specs,
        out_shape=out_shape,
    )


def _flatten_tc(x):
    def body(x_ref, o_ref):
        for c in range(CH):
            o_ref[c] = x_ref[:, c * CW:(c + 1) * CW]

    return pl.pallas_call(
        body,
        grid=(NBLK,),
        in_specs=[pl.BlockSpec((BN, D), lambda i: (i, 0))],
        out_specs=pl.BlockSpec((CH, BN, CW), lambda i: (0, i, 0)),
        out_shape=jax.ShapeDtypeStruct((CH, N, CW), jnp.float32),
    )(x)


_layer_relu = _make_layer_tc(D, D, "relu", want_flat=True)
_layer_lsm = _make_layer_tc(D, D, "logsoftmax", want_flat=False)


def kernel(x, edge_index, edge_type, basis0, comp0, root0, bias0,
           basis1, comp1, root1, bias1, basis2, comp2, root2, bias2):
    src = edge_index[0]
    dst = edge_index[1]
    eidx = (edge_type * N + dst).reshape(NSUB, NBATCH, K)
    offs = jnp.arange(CH, dtype=jnp.int32) * N
    src4 = (src[None, :] + offs[:, None]).reshape(CH, NSUB, NBATCH, K)
    zstripe = jnp.zeros((STRIPE, CW), jnp.float32)
    onesr = jnp.ones((K, CW), jnp.float32)

    xf = _flatten_tc(x).reshape(CH * N, CW)
    sums, counts = _sc_call(True, xf, src4, eidx, zstripe, onesr)
    cnt4a = counts[:RN, 0].reshape(R, NBLK, 1, BN)
    cnt4b = counts[RNP:RNP + RN, 0].reshape(R, NBLK, 1, BN)
    w = _wprep(comp0, basis0, D)
    h, hf = _layer_relu(x, sums, cnt4a, cnt4b, w, root0, bias0.reshape(1, D))

    sums, = _sc_call(False, hf.reshape(CH * N, CW), src4, eidx, zstripe,
                     onesr)
    w = _wprep(comp1, basis1, D)
    h, hf = _layer_relu(h, sums, cnt4a, cnt4b, w, root1, bias1.reshape(1, D))

    sums, = _sc_call(False, hf.reshape(CH * N, CW), src4, eidx, zstripe,
                     onesr)
    w = _wprep(comp2, basis2, 2)
    wp = jnp.pad(w, ((0, 0), (0, 0), (0, D - 2)))
    rootp = jnp.pad(root2, ((0, 0), (0, D - 2)))
    biasp = jnp.pad(bias2, (0, D - 2)).reshape(1, D)
    out, = _layer_lsm(h, sums, cnt4a, cnt4b, wp, rootp, biasp)
    return out[:, :2]


# speedup vs baseline: 14.0170x; 1.1479x over previous
import functools

import jax
import jax.numpy as jnp
from jax import lax
from jax.experimental import pallas as pl
from jax.experimental.pallas import tpu as pltpu
from jax.experimental.pallas import tpu_sc as plsc

N = 10000
E = 320000
D = 128
R = 6
NB = 30

CH = 8
CW = 16
RN = R * N
RNP = 60032
STRIPE = RNP // 16
NSUB = 16
NCORE = 2
ES = E // NSUB
K = 100
NBATCH = ES // K

BN = 1000
NBLK = N // BN


SB = 5
NSB = NBATCH // SB


@functools.lru_cache(maxsize=None)
def _make_sc_agg(with_counts):
    out_type = [jax.ShapeDtypeStruct((RNP, D), jnp.float32)]
    if with_counts:
        out_type.append(
            jax.ShapeDtypeStruct((NCORE * RNP, CW), jnp.float32))
    mesh = plsc.VectorSubcoreMesh(core_axis_name="c", subcore_axis_name="s")

    @functools.partial(
        pl.kernel,
        out_type=out_type,
        mesh=mesh,
        compiler_params=pltpu.CompilerParams(use_tc_tiling_on_sc=False),
        scratch_types=[
            pltpu.VMEM((NBATCH, K), jnp.int32),
            pltpu.VMEM((NBATCH, K), jnp.int32),
            pltpu.VMEM((2 * SB, K, CW), jnp.float32),
            pltpu.VMEM_SHARED((RNP, CW), jnp.float32),
            pltpu.SemaphoreType.DMA((2 * SB,)),
            pltpu.SemaphoreType.DMA((2,)),
        ],
    )
    def sc_agg(xflat, src4, eidx, zstripe, onesr, *out_and_scratch):
        if with_counts:
            out, cout = out_and_scratch[:2]
            eidx_vm, src_vm, rowbuf, acc, gsems, ssems = out_and_scratch[2:]
        else:
            out = out_and_scratch[0]
            eidx_vm, src_vm, rowbuf, acc, gsems, ssems = out_and_scratch[1:]
        core = lax.axis_index("c")
        sub = lax.axis_index("s")
        row0 = sub * STRIPE
        pltpu.sync_copy(eidx.at[sub], eidx_vm)
        for c in range(CH):
            @pl.when(core == (c % NCORE))
            def _(c=c):
                pltpu.sync_copy(zstripe, acc.at[pl.ds(row0, STRIPE)])
                pltpu.sync_copy(src4.at[c].at[sub], src_vm)
                plsc.subcore_barrier()

                for j in range(SB):
                    pltpu.async_copy(xflat.at[src_vm.at[j]],
                                     rowbuf.at[j], gsems.at[j])

                def step(t, carry):
                    p = t % 2
                    ob = 1 - p
                    bb = t * SB

                    @pl.when(t >= 1)
                    def _():
                        for j in range(SB):
                            pltpu.make_async_copy(
                                rowbuf.at[ob * SB + j],
                                acc.at[eidx_vm.at[bb - SB + j]],
                                ssems.at[ob]).wait()

                    @pl.when(t < NSB - 1)
                    def _():
                        for j in range(SB):
                            pltpu.async_copy(
                                xflat.at[src_vm.at[bb + SB + j]],
                                rowbuf.at[ob * SB + j],
                                gsems.at[ob * SB + j])

                    for j in range(SB):
                        pltpu.make_async_copy(
                            xflat.at[src_vm.at[bb + j]],
                            rowbuf.at[p * SB + j],
                            gsems.at[p * SB + j]).wait()
                        pltpu.async_copy(rowbuf.at[p * SB + j],
                                         acc.at[eidx_vm.at[bb + j]],
                                         ssems.at[p], add=True)
                    return carry

                lax.fori_loop(0, NSB, step, 0)
                for j in range(SB):
                    pltpu.make_async_copy(
                        rowbuf.at[SB + j],
                        acc.at[eidx_vm.at[NBATCH - SB + j]],
                        ssems.at[1]).wait()
                plsc.subcore_barrier()
                pltpu.sync_copy(
                    acc.at[pl.ds(row0, STRIPE)],
                    out.at[pl.ds(row0, STRIPE), pl.ds(c * CW, CW)])
                plsc.subcore_barrier()
        if with_counts:
            pltpu.sync_copy(zstripe, acc.at[pl.ds(row0, STRIPE)])
            pltpu.sync_copy(onesr, rowbuf.at[0])
            plsc.subcore_barrier()
            half = NBATCH // NCORE
            b0 = core * half

            def cstep(i, carry):
                pltpu.sync_copy(rowbuf.at[0],
                                acc.at[eidx_vm.at[b0 + i]], add=True)
                return carry

            lax.fori_loop(0, half, cstep, 0)
            plsc.subcore_barrier()
            pltpu.sync_copy(acc.at[pl.ds(row0, STRIPE)],
                            cout.at[pl.ds(core * RNP + row0, STRIPE)])

    return sc_agg


def _sc_call(with_counts, *args):
    return _make_sc_agg(with_counts)(*args)


def _wprep(comp, basis, dout):
    din = basis.shape[1]

    def body(c_ref, b_ref, o_ref):
        o_ref[...] = jnp.dot(c_ref[...], b_ref[...],
                             preferred_element_type=jnp.float32)

    flat = pl.pallas_call(
        body,
        out_shape=jax.ShapeDtypeStruct((R, din * dout), jnp.float32),
    )(comp, basis.reshape(NB, din * dout))
    return flat.reshape(R, din, dout)


def _make_layer_tc(din, dout, act, want_flat):
    def body(x_ref, acc_ref, cnta_ref, cntb_ref, w_ref, root_ref, bias_ref,
             *o_refs):
        o_ref = o_refs[0]
        r = pl.program_id(1)
        contrib = jnp.dot(acc_ref[...], w_ref[0],
                          preferred_element_type=jnp.float32)
        cnt = jnp.maximum(cnta_ref[0, 0, 0, :] + cntb_ref[0, 0, 0, :], 1.0)
        contrib = contrib / cnt[:, None]

        @pl.when(r == 0)
        def _():
            o_ref[...] = (jnp.dot(x_ref[...], root_ref[...],
                                  preferred_element_type=jnp.float32)
                          + bias_ref[...] + contrib)

        @pl.when(r > 0)
        def _():
            o_ref[...] = o_ref[...] + contrib

        @pl.when(r == R - 1)
        def _():
            h = o_ref[...]
            if act == "relu":
                h = jnp.maximum(h, 0.0)
                o_ref[...] = h
            else:
                col = lax.broadcasted_iota(jnp.int32, (BN, dout), 1)
                hm = jnp.where(col < 2, h, -jnp.inf)
                m = jnp.max(hm, axis=1, keepdims=True)
                lse = m + jnp.log(jnp.sum(jnp.exp(hm - m), axis=1,
                                          keepdims=True))
                o_ref[...] = h - lse
            if want_flat:
                for c in range(CH):
                    o_refs[1][c] = h[:, c * CW:(c + 1) * CW]

    out_specs = [pl.BlockSpec((BN, dout), lambda i, r: (i, 0))]
    out_shape = [jax.ShapeDtypeStruct((N, dout), jnp.float32)]
    if want_flat:
        out_specs.append(pl.BlockSpec((CH, BN, CW), lambda i, r: (0, i, 0)))
        out_shape.append(jax.ShapeDtypeStruct((CH, N, CW), jnp.float32))
    return pl.pallas_call(
        body,
        grid=(NBLK, R),
        in_specs=[
            pl.BlockSpec((BN, din), lambda i, r: (i, 0)),
            pl.BlockSpec((BN, din), lambda i, r: (r * NBLK + i, 0)),
            pl.BlockSpec((1, 1, 1, BN), lambda i, r: (r, i, 0, 0)),
            pl.BlockSpec((1, 1, 1, BN), lambda i, r: (r, i, 0, 0)),
            pl.BlockSpec((1, din, dout), lambda i, r: (r, 0, 0)),
            pl.BlockSpec((din, dout), lambda i, r: (0, 0)),
            pl.BlockSpec((1, dout), lambda i, r: (0, 0)),
        ],
        out_specs=out_specs,
        out_shape=out_shape,
    )


def _flatten_tc(x):
    def body(x_ref, o_ref):
        for c in range(CH):
            o_ref[c] = x_ref[:, c * CW:(c + 1) * CW]

    return pl.pallas_call(
        body,
        grid=(NBLK,),
        in_specs=[pl.BlockSpec((BN, D), lambda i: (i, 0))],
        out_specs=pl.BlockSpec((CH, BN, CW), lambda i: (0, i, 0)),
        out_shape=jax.ShapeDtypeStruct((CH, N, CW), jnp.float32),
    )(x)


_layer_relu = _make_layer_tc(D, D, "relu", want_flat=True)
_layer_lsm = _make_layer_tc(D, D, "logsoftmax", want_flat=False)


def kernel(x, edge_index, edge_type, basis0, comp0, root0, bias0,
           basis1, comp1, root1, bias1, basis2, comp2, root2, bias2):
    src = edge_index[0]
    dst = edge_index[1]
    eidx = (edge_type * N + dst).reshape(NSUB, NBATCH, K)
    offs = jnp.arange(CH, dtype=jnp.int32) * N
    src4 = (src[None, :] + offs[:, None]).reshape(CH, NSUB, NBATCH, K)
    zstripe = jnp.zeros((STRIPE, CW), jnp.float32)
    onesr = jnp.ones((K, CW), jnp.float32)

    xf = _flatten_tc(x).reshape(CH * N, CW)
    sums, counts = _sc_call(True, xf, src4, eidx, zstripe, onesr)
    cnt4a = counts[:RN, 0].reshape(R, NBLK, 1, BN)
    cnt4b = counts[RNP:RNP + RN, 0].reshape(R, NBLK, 1, BN)
    w = _wprep(comp0, basis0, D)
    h, hf = _layer_relu(x, sums, cnt4a, cnt4b, w, root0, bias0.reshape(1, D))

    sums, = _sc_call(False, hf.reshape(CH * N, CW), src4, eidx, zstripe,
                     onesr)
    w = _wprep(comp1, basis1, D)
    h, hf = _layer_relu(h, sums, cnt4a, cnt4b, w, root1, bias1.reshape(1, D))

    sums, = _sc_call(False, hf.reshape(CH * N, CW), src4, eidx, zstripe,
                     onesr)
    w = _wprep(comp2, basis2, 2)
    wp = jnp.pad(w, ((0, 0), (0, 0), (0, D - 2)))
    rootp = jnp.pad(root2, ((0, 0), (0, D - 2)))
    biasp = jnp.pad(bias2, (0, D - 2)).reshape(1, D)
    out, = _layer_lsm(h, sums, cnt4a, cnt4b, wp, rootp, biasp)
    return out[:, :2]


# speedup vs baseline: 14.5254x; 1.0363x over previous
import functools

import jax
import jax.numpy as jnp
from jax import lax
from jax.experimental import pallas as pl
from jax.experimental.pallas import tpu as pltpu
from jax.experimental.pallas import tpu_sc as plsc

N = 10000
E = 320000
D = 128
R = 6
NB = 30

CH = 8
CW = 16
RN = R * N
RNP = 60032
STRIPE = RNP // 16
NSUB = 16
NCORE = 2
ES = E // NSUB
K = 100
NBATCH = ES // K

BN = 1000
NBLK = N // BN


SB = 5
NSB = NBATCH // SB


@functools.lru_cache(maxsize=None)
def _make_sc_agg(with_counts):
    out_type = [jax.ShapeDtypeStruct((RNP, D), jnp.float32)]
    if with_counts:
        out_type.append(
            jax.ShapeDtypeStruct((NCORE * RNP, CW), jnp.float32))
    mesh = plsc.VectorSubcoreMesh(core_axis_name="c", subcore_axis_name="s")

    @functools.partial(
        pl.kernel,
        out_type=out_type,
        mesh=mesh,
        compiler_params=pltpu.CompilerParams(use_tc_tiling_on_sc=False),
        scratch_types=[
            pltpu.VMEM((NBATCH, K), jnp.int32),
            pltpu.VMEM((NBATCH, K), jnp.int32),
            pltpu.VMEM((2 * SB, K, CW), jnp.float32),
            pltpu.VMEM_SHARED((RNP, CW), jnp.float32),
            pltpu.SemaphoreType.DMA((2 * SB,)),
            pltpu.SemaphoreType.DMA((2,)),
        ],
    )
    def sc_agg(x3, srcr, eidx, zstripe, onesr, *out_and_scratch):
        if with_counts:
            out, cout = out_and_scratch[:2]
            eidx_vm, src_vm, rowbuf, acc, gsems, ssems = out_and_scratch[2:]
        else:
            out = out_and_scratch[0]
            eidx_vm, src_vm, rowbuf, acc, gsems, ssems = out_and_scratch[1:]
        core = lax.axis_index("c")
        sub = lax.axis_index("s")
        row0 = sub * STRIPE
        pltpu.sync_copy(eidx.at[sub], eidx_vm)
        pltpu.sync_copy(srcr.at[sub], src_vm)
        for c in range(CH):
            @pl.when(core == (c % NCORE))
            def _(c=c):
                xflat = x3.at[c]
                pltpu.sync_copy(zstripe, acc.at[pl.ds(row0, STRIPE)])
                plsc.subcore_barrier()

                for j in range(SB):
                    pltpu.async_copy(xflat.at[src_vm.at[j]],
                                     rowbuf.at[j], gsems.at[j])

                def step(t, carry):
                    p = t % 2
                    ob = 1 - p
                    bb = t * SB

                    @pl.when(t >= 1)
                    def _():
                        for j in range(SB):
                            pltpu.make_async_copy(
                                rowbuf.at[ob * SB + j],
                                acc.at[eidx_vm.at[bb - SB + j]],
                                ssems.at[ob]).wait()

                    @pl.when(t < NSB - 1)
                    def _():
                        for j in range(SB):
                            pltpu.async_copy(
                                xflat.at[src_vm.at[bb + SB + j]],
                                rowbuf.at[ob * SB + j],
                                gsems.at[ob * SB + j])

                    for j in range(SB):
                        pltpu.make_async_copy(
                            xflat.at[src_vm.at[bb + j]],
                            rowbuf.at[p * SB + j],
                            gsems.at[p * SB + j]).wait()
                        pltpu.async_copy(rowbuf.at[p * SB + j],
                                         acc.at[eidx_vm.at[bb + j]],
                                         ssems.at[p], add=True)
                    return carry

                lax.fori_loop(0, NSB, step, 0)
                for j in range(SB):
                    pltpu.make_async_copy(
                        rowbuf.at[SB + j],
                        acc.at[eidx_vm.at[NBATCH - SB + j]],
                        ssems.at[1]).wait()
                plsc.subcore_barrier()
                pltpu.sync_copy(
                    acc.at[pl.ds(row0, STRIPE)],
                    out.at[pl.ds(row0, STRIPE), pl.ds(c * CW, CW)])
                plsc.subcore_barrier()
        if with_counts:
            pltpu.sync_copy(zstripe, acc.at[pl.ds(row0, STRIPE)])
            pltpu.sync_copy(onesr, rowbuf.at[0])
            plsc.subcore_barrier()
            half = NBATCH // NCORE
            b0 = core * half

            def cstep(i, carry):
                pltpu.sync_copy(rowbuf.at[0],
                                acc.at[eidx_vm.at[b0 + i]], add=True)
                return carry

            lax.fori_loop(0, half, cstep, 0)
            plsc.subcore_barrier()
            pltpu.sync_copy(acc.at[pl.ds(row0, STRIPE)],
                            cout.at[pl.ds(core * RNP + row0, STRIPE)])

    return sc_agg


def _sc_call(with_counts, *args):
    return _make_sc_agg(with_counts)(*args)


def _wprep(comp, basis, dout):
    din = basis.shape[1]

    def body(c_ref, b_ref, o_ref):
        o_ref[...] = jnp.dot(c_ref[...], b_ref[...],
                             preferred_element_type=jnp.float32)

    flat = pl.pallas_call(
        body,
        out_shape=jax.ShapeDtypeStruct((R, din * dout), jnp.float32),
    )(comp, basis.reshape(NB, din * dout))
    return flat.reshape(R, din, dout)


def _make_layer_tc(din, dout, act, want_flat):
    def body(x_ref, acc_ref, cnta_ref, cntb_ref, w_ref, root_ref, bias_ref,
             *o_refs):
        o_ref = o_refs[0]
        r = pl.program_id(1)
        contrib = jnp.dot(acc_ref[...], w_ref[0],
                          preferred_element_type=jnp.float32)
        cnt = jnp.maximum(cnta_ref[0, 0, 0, :] + cntb_ref[0, 0, 0, :], 1.0)
        contrib = contrib / cnt[:, None]

        @pl.when(r == 0)
        def _():
            o_ref[...] = (jnp.dot(x_ref[...], root_ref[...],
                                  preferred_element_type=jnp.float32)
                          + bias_ref[...] + contrib)

        @pl.when(r > 0)
        def _():
            o_ref[...] = o_ref[...] + contrib

        @pl.when(r == R - 1)
        def _():
            h = o_ref[...]
            if act == "relu":
                h = jnp.maximum(h, 0.0)
                o_ref[...] = h
            else:
                col = lax.broadcasted_iota(jnp.int32, (BN, dout), 1)
                hm = jnp.where(col < 2, h, -jnp.inf)
                m = jnp.max(hm, axis=1, keepdims=True)
                lse = m + jnp.log(jnp.sum(jnp.exp(hm - m), axis=1,
                                          keepdims=True))
                o_ref[...] = h - lse
            if want_flat:
                for c in range(CH):
                    o_refs[1][c] = h[:, c * CW:(c + 1) * CW]

    out_specs = [pl.BlockSpec((BN, dout), lambda i, r: (i, 0))]
    out_shape = [jax.ShapeDtypeStruct((N, dout), jnp.float32)]
    if want_flat:
        out_specs.append(pl.BlockSpec((CH, BN, CW), lambda i, r: (0, i, 0)))
        out_shape.append(jax.ShapeDtypeStruct((CH, N, CW), jnp.float32))
    return pl.pallas_call(
        body,
        grid=(NBLK, R),
        in_specs=[
            pl.BlockSpec((BN, din), lambda i, r: (i, 0)),
            pl.BlockSpec((BN, din), lambda i, r: (r * NBLK + i, 0)),
            pl.BlockSpec((1, 1, 1, BN), lambda i, r: (r, i, 0, 0)),
            pl.BlockSpec((1, 1, 1, BN), lambda i, r: (r, i, 0, 0)),
            pl.BlockSpec((1, din, dout), lambda i, r: (r, 0, 0)),
            pl.BlockSpec((din, dout), lambda i, r: (0, 0)),
            pl.BlockSpec((1, dout), lambda i, r: (0, 0)),
        ],
        out_specs=out_specs,
        out_shape=out_shape,
    )


def _flatten_tc(x):
    def body(x_ref, o_ref):
        for c in range(CH):
            o_ref[c] = x_ref[:, c * CW:(c + 1) * CW]

    return pl.pallas_call(
        body,
        grid=(NBLK,),
        in_specs=[pl.BlockSpec((BN, D), lambda i: (i, 0))],
        out_specs=pl.BlockSpec((CH, BN, CW), lambda i: (0, i, 0)),
        out_shape=jax.ShapeDtypeStruct((CH, N, CW), jnp.float32),
    )(x)


_layer_relu = _make_layer_tc(D, D, "relu", want_flat=True)
_layer_lsm = _make_layer_tc(D, D, "logsoftmax", want_flat=False)


def kernel(x, edge_index, edge_type, basis0, comp0, root0, bias0,
           basis1, comp1, root1, bias1, basis2, comp2, root2, bias2):
    src = edge_index[0]
    dst = edge_index[1]
    eidx = (edge_type * N + dst).reshape(NSUB, NBATCH, K)
    srcr = src.reshape(NSUB, NBATCH, K)
    zstripe = jnp.zeros((STRIPE, CW), jnp.float32)
    onesr = jnp.ones((K, CW), jnp.float32)

    sums, counts = _sc_call(True, _flatten_tc(x), srcr, eidx, zstripe, onesr)
    cnt4a = counts[:RN, 0].reshape(R, NBLK, 1, BN)
    cnt4b = counts[RNP:RNP + RN, 0].reshape(R, NBLK, 1, BN)
    w = _wprep(comp0, basis0, D)
    h, hf = _layer_relu(x, sums, cnt4a, cnt4b, w, root0, bias0.reshape(1, D))

    sums, = _sc_call(False, hf, srcr, eidx, zstripe, onesr)
    w = _wprep(comp1, basis1, D)
    h, hf = _layer_relu(h, sums, cnt4a, cnt4b, w, root1, bias1.reshape(1, D))

    sums, = _sc_call(False, hf, srcr, eidx, zstripe, onesr)
    w = _wprep(comp2, basis2, 2)
    wp = jnp.pad(w, ((0, 0), (0, 0), (0, D - 2)))
    rootp = jnp.pad(root2, ((0, 0), (0, D - 2)))
    biasp = jnp.pad(bias2, (0, D - 2)).reshape(1, D)
    out, = _layer_lsm(h, sums, cnt4a, cnt4b, wp, rootp, biasp)
    return out[:, :2]


# speedup vs baseline: 15.1121x; 1.0404x over previous
import functools

import jax
import jax.numpy as jnp
from jax import lax
from jax.experimental import pallas as pl
from jax.experimental.pallas import tpu as pltpu
from jax.experimental.pallas import tpu_sc as plsc

N = 10000
E = 320000
D = 128
R = 6
NB = 30

CH = 8
CW = 16
RN = R * N
RNP = 60032
STRIPE = RNP // 16
NSUB = 16
NCORE = 2
ES = E // NSUB
K = 125
NBATCH = ES // K

BN = 1000
NBLK = N // BN


SB = 5
NSB = NBATCH // SB


@functools.lru_cache(maxsize=None)
def _make_sc_agg(with_counts):
    out_type = [jax.ShapeDtypeStruct((RNP, D), jnp.float32)]
    if with_counts:
        out_type.append(
            jax.ShapeDtypeStruct((NCORE * RNP, CW), jnp.float32))
    mesh = plsc.VectorSubcoreMesh(core_axis_name="c", subcore_axis_name="s")

    @functools.partial(
        pl.kernel,
        out_type=out_type,
        mesh=mesh,
        compiler_params=pltpu.CompilerParams(use_tc_tiling_on_sc=False),
        scratch_types=[
            pltpu.VMEM((NBATCH, K), jnp.int32),
            pltpu.VMEM((NBATCH, K), jnp.int32),
            pltpu.VMEM((2 * SB, K, CW), jnp.float32),
            pltpu.VMEM_SHARED((RNP, CW), jnp.float32),
            pltpu.SemaphoreType.DMA((2 * SB,)),
            pltpu.SemaphoreType.DMA((2,)),
        ],
    )
    def sc_agg(x3, srcr, eidx, zstripe, onesr, *out_and_scratch):
        if with_counts:
            out, cout = out_and_scratch[:2]
            eidx_vm, src_vm, rowbuf, acc, gsems, ssems = out_and_scratch[2:]
        else:
            out = out_and_scratch[0]
            eidx_vm, src_vm, rowbuf, acc, gsems, ssems = out_and_scratch[1:]
        core = lax.axis_index("c")
        sub = lax.axis_index("s")
        row0 = sub * STRIPE
        pltpu.sync_copy(eidx.at[sub], eidx_vm)
        pltpu.sync_copy(srcr.at[sub], src_vm)
        for c in range(CH):
            @pl.when(core == (c % NCORE))
            def _(c=c):
                xflat = x3.at[c]
                pltpu.sync_copy(zstripe, acc.at[pl.ds(row0, STRIPE)])
                plsc.subcore_barrier()

                for j in range(SB):
                    pltpu.async_copy(xflat.at[src_vm.at[j]],
                                     rowbuf.at[j], gsems.at[j])

                def step(t, carry):
                    p = t % 2
                    ob = 1 - p
                    bb = t * SB

                    @pl.when(t >= 1)
                    def _():
                        for j in range(SB):
                            pltpu.make_async_copy(
                                rowbuf.at[ob * SB + j],
                                acc.at[eidx_vm.at[bb - SB + j]],
                                ssems.at[ob]).wait()

                    @pl.when(t < NSB - 1)
                    def _():
                        for j in range(SB):
                            pltpu.async_copy(
                                xflat.at[src_vm.at[bb + SB + j]],
                                rowbuf.at[ob * SB + j],
                                gsems.at[ob * SB + j])

                    for j in range(SB):
                        pltpu.make_async_copy(
                            xflat.at[src_vm.at[bb + j]],
                            rowbuf.at[p * SB + j],
                            gsems.at[p * SB + j]).wait()
                        pltpu.async_copy(rowbuf.at[p * SB + j],
                                         acc.at[eidx_vm.at[bb + j]],
                                         ssems.at[p], add=True)
                    return carry

                lax.fori_loop(0, NSB, step, 0)
                for j in range(SB):
                    pltpu.make_async_copy(
                        rowbuf.at[SB + j],
                        acc.at[eidx_vm.at[NBATCH - SB + j]],
                        ssems.at[1]).wait()
                plsc.subcore_barrier()
                pltpu.sync_copy(
                    acc.at[pl.ds(row0, STRIPE)],
                    out.at[pl.ds(row0, STRIPE), pl.ds(c * CW, CW)])
                plsc.subcore_barrier()
        if with_counts:
            pltpu.sync_copy(zstripe, acc.at[pl.ds(row0, STRIPE)])
            pltpu.sync_copy(onesr, rowbuf.at[0])
            plsc.subcore_barrier()
            half = NBATCH // NCORE
            b0 = core * half

            def cstep(i, carry):
                pltpu.sync_copy(rowbuf.at[0],
                                acc.at[eidx_vm.at[b0 + i]], add=True)
                return carry

            lax.fori_loop(0, half, cstep, 0)
            plsc.subcore_barrier()
            pltpu.sync_copy(acc.at[pl.ds(row0, STRIPE)],
                            cout.at[pl.ds(core * RNP + row0, STRIPE)])

    return sc_agg


def _sc_call(with_counts, *args):
    return _make_sc_agg(with_counts)(*args)


def _wprep(comp, basis, dout):
    din = basis.shape[1]

    def body(c_ref, b_ref, o_ref):
        o_ref[...] = jnp.dot(c_ref[...], b_ref[...],
                             preferred_element_type=jnp.float32)

    flat = pl.pallas_call(
        body,
        out_shape=jax.ShapeDtypeStruct((R, din * dout), jnp.float32),
    )(comp, basis.reshape(NB, din * dout))
    return flat.reshape(R, din, dout)


def _make_layer_tc(din, dout, act, want_flat):
    def body(x_ref, acc_ref, cnta_ref, cntb_ref, w_ref, root_ref, bias_ref,
             *o_refs):
        o_ref = o_refs[0]
        r = pl.program_id(1)
        contrib = jnp.dot(acc_ref[...], w_ref[0],
                          preferred_element_type=jnp.float32)
        cnt = jnp.maximum(cnta_ref[0, 0, 0, :] + cntb_ref[0, 0, 0, :], 1.0)
        contrib = contrib / cnt[:, None]

        @pl.when(r == 0)
        def _():
            o_ref[...] = (jnp.dot(x_ref[...], root_ref[...],
                                  preferred_element_type=jnp.float32)
                          + bias_ref[...] + contrib)

        @pl.when(r > 0)
        def _():
            o_ref[...] = o_ref[...] + contrib

        @pl.when(r == R - 1)
        def _():
            h = o_ref[...]
            if act == "relu":
                h = jnp.maximum(h, 0.0)
                o_ref[...] = h
            else:
                col = lax.broadcasted_iota(jnp.int32, (BN, dout), 1)
                hm = jnp.where(col < 2, h, -jnp.inf)
                m = jnp.max(hm, axis=1, keepdims=True)
                lse = m + jnp.log(jnp.sum(jnp.exp(hm - m), axis=1,
                                          keepdims=True))
                o_ref[...] = h - lse
            if want_flat:
                for c in range(CH):
                    o_refs[1][c] = h[:, c * CW:(c + 1) * CW]

    out_specs = [pl.BlockSpec((BN, dout), lambda i, r: (i, 0))]
    out_shape = [jax.ShapeDtypeStruct((N, dout), jnp.float32)]
    if want_flat:
        out_specs.append(pl.BlockSpec((CH, BN, CW), lambda i, r: (0, i, 0)))
        out_shape.append(jax.ShapeDtypeStruct((CH, N, CW), jnp.float32))
    return pl.pallas_call(
        body,
        grid=(NBLK, R),
        in_specs=[
            pl.BlockSpec((BN, din), lambda i, r: (i, 0)),
            pl.BlockSpec((BN, din), lambda i, r: (r * NBLK + i, 0)),
            pl.BlockSpec((1, 1, 1, BN), lambda i, r: (r, i, 0, 0)),
            pl.BlockSpec((1, 1, 1, BN), lambda i, r: (r, i, 0, 0)),
            pl.BlockSpec((1, din, dout), lambda i, r: (r, 0, 0)),
            pl.BlockSpec((din, dout), lambda i, r: (0, 0)),
            pl.BlockSpec((1, dout), lambda i, r: (0, 0)),
        ],
        out_specs=out_specs,
        out_shape=out_shape,
    )


def _flatten_tc(x):
    def body(x_ref, o_ref):
        for c in range(CH):
            o_ref[c] = x_ref[:, c * CW:(c + 1) * CW]

    return pl.pallas_call(
        body,
        grid=(NBLK,),
        in_specs=[pl.BlockSpec((BN, D), lambda i: (i, 0))],
        out_specs=pl.BlockSpec((CH, BN, CW), lambda i: (0, i, 0)),
        out_shape=jax.ShapeDtypeStruct((CH, N, CW), jnp.float32),
    )(x)


_layer_relu = _make_layer_tc(D, D, "relu", want_flat=True)
_layer_lsm = _make_layer_tc(D, D, "logsoftmax", want_flat=False)


def kernel(x, edge_index, edge_type, basis0, comp0, root0, bias0,
           basis1, comp1, root1, bias1, basis2, comp2, root2, bias2):
    src = edge_index[0]
    dst = edge_index[1]
    eidx = (edge_type * N + dst).reshape(NSUB, NBATCH, K)
    srcr = src.reshape(NSUB, NBATCH, K)
    zstripe = jnp.zeros((STRIPE, CW), jnp.float32)
    onesr = jnp.ones((K, CW), jnp.float32)

    sums, counts = _sc_call(True, _flatten_tc(x), srcr, eidx, zstripe, onesr)
    cnt4a = counts[:RN, 0].reshape(R, NBLK, 1, BN)
    cnt4b = counts[RNP:RNP + RN, 0].reshape(R, NBLK, 1, BN)
    w = _wprep(comp0, basis0, D)
    h, hf = _layer_relu(x, sums, cnt4a, cnt4b, w, root0, bias0.reshape(1, D))

    sums, = _sc_call(False, hf, srcr, eidx, zstripe, onesr)
    w = _wprep(comp1, basis1, D)
    h, hf = _layer_relu(h, sums, cnt4a, cnt4b, w, root1, bias1.reshape(1, D))

    sums, = _sc_call(False, hf, srcr, eidx, zstripe, onesr)
    w = _wprep(comp2, basis2, 2)
    wp = jnp.pad(w, ((0, 0), (0, 0), (0, D - 2)))
    rootp = jnp.pad(root2, ((0, 0), (0, D - 2)))
    biasp = jnp.pad(bias2, (0, D - 2)).reshape(1, D)
    out, = _layer_lsm(h, sums, cnt4a, cnt4b, wp, rootp, biasp)
    return out[:, :2]
